# baseline (speedup 1.0000x reference)
import jax
import jax.numpy as jnp
from jax import lax
from jax.experimental import pallas as pl
from jax.experimental.pallas import tpu as pltpu

N = 1024
ED = 2048
K = 4
BIG = 1e30


def _bdotT(x, w):
    return lax.dot_general(x, w, (((1,), (1,)), ((), ())),
                           preferred_element_type=jnp.float32)


def _encode_kernel(a_ref, t_ref, v_ref, wa_ref, ba_ref, wt_ref, bt_ref,
                   wv_ref, bv_ref, eab_ref, etb_ref, evb_ref,
                   sqa_ref, sqt_ref, sqv_ref):
    ea = _bdotT(a_ref[...], wa_ref[...].astype(jnp.bfloat16)) + ba_ref[...]
    et = _bdotT(t_ref[...], wt_ref[...].astype(jnp.bfloat16)) + bt_ref[...]
    ev = _bdotT(v_ref[...], wv_ref[...].astype(jnp.bfloat16)) + bv_ref[...]
    eab_ref[...] = ea.astype(jnp.bfloat16)
    etb_ref[...] = et.astype(jnp.bfloat16)
    evb_ref[...] = ev.astype(jnp.bfloat16)
    sqa_ref[...] = jnp.sum(ea * ea, axis=1, keepdims=True)
    sqt_ref[...] = jnp.sum(et * et, axis=1, keepdims=True)
    sqv_ref[...] = jnp.sum(ev * ev, axis=1, keepdims=True)


def _affinity_kernel(x_ref, sq_ref, mrow_ref, mcol_ref, p_ref):
    sq = sq_ref[...]
    g = _bdotT(x_ref[...], x_ref[...])
    d2 = sq + sq.T - 2.0 * g
    iota = lax.broadcasted_iota(jnp.int32, (N, N), 1)
    rowi = lax.broadcasted_iota(jnp.int32, (N, N), 0)
    eye = iota == rowi
    jstars = []
    sims = []
    for s in range(K):
        excl = jnp.zeros((N, N), jnp.bool_)
        for j in jstars:
            excl = excl | (iota == j)
        deff = jnp.where(excl, BIG, d2)
        m = jnp.min(deff, axis=1, keepdims=True)
        jstar = jnp.min(jnp.where(deff == m, iota, N), axis=1, keepdims=True)
        jstars.append(jstar)
        sims.append(1.0 / (1.0 + jnp.sqrt(jnp.maximum(m, 0.0) + 1e-12)))
    a_mat = jnp.zeros((N, N), jnp.float32)
    for jstar, sim in zip(jstars, sims):
        a_mat = a_mat + jnp.where(iota == jstar, sim, 0.0)
    a_mat = a_mat * mrow_ref[...] * mcol_ref[...]
    a_mat = jnp.maximum(a_mat, a_mat.T)
    s_mat = jnp.where(eye, 2.0, a_mat)
    dc = lax.rsqrt(jnp.sum(s_mat, axis=1, keepdims=True) + 1e-12)
    p_ref[...] = (dc * s_mat * dc.T).astype(jnp.bfloat16)


def _fold_kernel(w_ref, d_ref, wb_ref, db_ref, m_ref, b2_ref):
    dbf = d_ref[...].astype(jnp.bfloat16)
    m_ref[...] = lax.dot_general(
        w_ref[...].astype(jnp.bfloat16), dbf, (((0,), (1,)), ((), ())),
        preferred_element_type=jnp.float32).astype(jnp.bfloat16)
    b2_ref[...] = lax.dot_general(
        wb_ref[...].astype(jnp.bfloat16), dbf, (((1,), (1,)), ((), ())),
        preferred_element_type=jnp.float32) + db_ref[...]


def _headmix_kernel(e_ref, m_ref, b2_ref, p1_ref, p2_ref, o_ref):
    y = lax.dot_general(e_ref[...], m_ref[...], (((1,), (0,)), ((), ())),
                        preferred_element_type=jnp.float32)
    mixed = lax.dot_general(p1_ref[...] + p2_ref[...], y.astype(jnp.bfloat16),
                            (((1,), (0,)), ((), ())),
                            preferred_element_type=jnp.float32)
    o_ref[...] = 0.5 * y + 0.25 * mixed + b2_ref[...]


def kernel(a, t, v, mask, Wa_w, Wa_b, Wt_w, Wt_b, Wv_w, Wv_b,
           wa_w, wa_b, wt_w, wt_b, wv_w, wv_b,
           da_w, da_b, dt_w, dt_b, dv_w, dv_b):
    f32 = jnp.float32
    bf16 = jnp.bfloat16
    mrow = mask.reshape(1, N)
    mcol = mask.reshape(N, 1)
    ab = a.astype(bf16)
    tb = t.astype(bf16)
    vb = v.astype(bf16)

    eab, etb, evb, sqa, sqt, sqv = pl.pallas_call(
        _encode_kernel,
        out_shape=[jax.ShapeDtypeStruct((N, ED), bf16)] * 3
        + [jax.ShapeDtypeStruct((N, 1), f32)] * 3,
    )(ab, tb, vb, Wa_w, Wa_b.reshape(1, -1), Wt_w, Wt_b.reshape(1, -1),
      Wv_w, Wv_b.reshape(1, -1))

    aff = pl.pallas_call(
        _affinity_kernel,
        out_shape=jax.ShapeDtypeStruct((N, N), bf16),
    )
    pa = aff(eab, sqa, mrow, mcol)
    pt = aff(etb, sqt, mrow, mcol)
    pv = aff(evb, sqv, mrow, mcol)

    def fold(w, wb, d, db):
        dout = d.shape[0]
        return pl.pallas_call(
            _fold_kernel,
            out_shape=[jax.ShapeDtypeStruct((ED, dout), bf16),
                       jax.ShapeDtypeStruct((1, dout), f32)],
        )(w, d, wb.reshape(1, -1), db.reshape(1, -1))

    ma, b2a = fold(wa_w, wa_b, da_w, da_b)
    mt, b2t = fold(wt_w, wt_b, dt_w, dt_b)
    mv, b2v = fold(wv_w, wv_b, dv_w, dv_b)

    def headmix(eb, m, b2, p1, p2):
        return pl.pallas_call(
            _headmix_kernel,
            out_shape=jax.ShapeDtypeStruct((N, m.shape[1]), f32),
        )(eb, m, b2, p1, p2)

    ra = headmix(eab, ma, b2a, pt, pv)
    rt = headmix(etb, mt, b2t, pv, pa)
    rv = headmix(evb, mv, b2v, pa, pt)
    return jnp.concatenate([ra, rt, rv], axis=1)


# speedup vs baseline: 1.0149x; 1.0149x over previous
import jax
import jax.numpy as jnp
from jax import lax
from jax.experimental import pallas as pl
from jax.experimental.pallas import tpu as pltpu

N = 1024
ED = 2048
K = 4
BIG = 1e30


def _bdotT(x, w):
    return lax.dot_general(x, w, (((1,), (1,)), ((), ())),
                           preferred_element_type=jnp.float32)


def _encode_kernel(a_ref, t_ref, v_ref, wa_ref, ba_ref, wt_ref, bt_ref,
                   wv_ref, bv_ref, eab_ref, etb_ref, evb_ref,
                   sqa_ref, sqt_ref, sqv_ref):
    ea = _bdotT(a_ref[...], wa_ref[...].astype(jnp.bfloat16)) + ba_ref[...]
    et = _bdotT(t_ref[...], wt_ref[...].astype(jnp.bfloat16)) + bt_ref[...]
    ev = _bdotT(v_ref[...], wv_ref[...].astype(jnp.bfloat16)) + bv_ref[...]
    eab_ref[...] = ea.astype(jnp.bfloat16)
    etb_ref[...] = et.astype(jnp.bfloat16)
    evb_ref[...] = ev.astype(jnp.bfloat16)
    sqa_ref[...] = jnp.sum(ea * ea, axis=1, keepdims=True)
    sqt_ref[...] = jnp.sum(et * et, axis=1, keepdims=True)
    sqv_ref[...] = jnp.sum(ev * ev, axis=1, keepdims=True)


def _affinity_kernel(x_ref, sq_ref, mrow_ref, mcol_ref, p_ref):
    sq = sq_ref[...]
    g = _bdotT(x_ref[...], x_ref[...])
    d2 = sq + sq.T - 2.0 * g
    iota = lax.broadcasted_iota(jnp.int32, (N, N), 1)
    rowi = lax.broadcasted_iota(jnp.int32, (N, N), 0)
    eye = iota == rowi
    jstars = []
    sims = []
    dwork = d2
    for _ in range(K):
        m = jnp.min(dwork, axis=1, keepdims=True)
        jstar = jnp.min(jnp.where(dwork == m, iota, N), axis=1, keepdims=True)
        jstars.append(jstar)
        sims.append(1.0 / (1.0 + jnp.sqrt(jnp.maximum(m, 0.0) + 1e-12)))
        dwork = jnp.where(iota == jstar, BIG, dwork)
    a_mat = jnp.zeros((N, N), jnp.float32)
    for jstar, sim in zip(jstars, sims):
        a_mat = a_mat + jnp.where(iota == jstar, sim, 0.0)
    a_mat = a_mat * mrow_ref[...] * mcol_ref[...]
    a_mat = jnp.maximum(a_mat, a_mat.T)
    s_mat = jnp.where(eye, 2.0, a_mat)
    dc = lax.rsqrt(jnp.sum(s_mat, axis=1, keepdims=True) + 1e-12)
    p_ref[...] = (dc * s_mat * dc.T).astype(jnp.bfloat16)


def _fold_kernel(w_ref, d_ref, wb_ref, db_ref, m_ref, b2_ref):
    dbf = d_ref[...].astype(jnp.bfloat16)
    m_ref[...] = lax.dot_general(
        w_ref[...].astype(jnp.bfloat16), dbf, (((0,), (1,)), ((), ())),
        preferred_element_type=jnp.float32).astype(jnp.bfloat16)
    b2_ref[...] = lax.dot_general(
        wb_ref[...].astype(jnp.bfloat16), dbf, (((1,), (1,)), ((), ())),
        preferred_element_type=jnp.float32) + db_ref[...]


def _headmix_kernel(e_ref, m_ref, b2_ref, p1_ref, p2_ref, o_ref):
    y = lax.dot_general(e_ref[...], m_ref[...], (((1,), (0,)), ((), ())),
                        preferred_element_type=jnp.float32)
    mixed = lax.dot_general(p1_ref[...] + p2_ref[...], y.astype(jnp.bfloat16),
                            (((1,), (0,)), ((), ())),
                            preferred_element_type=jnp.float32)
    o_ref[...] = 0.5 * y + 0.25 * mixed + b2_ref[...]


def kernel(a, t, v, mask, Wa_w, Wa_b, Wt_w, Wt_b, Wv_w, Wv_b,
           wa_w, wa_b, wt_w, wt_b, wv_w, wv_b,
           da_w, da_b, dt_w, dt_b, dv_w, dv_b):
    f32 = jnp.float32
    bf16 = jnp.bfloat16
    mrow = mask.reshape(1, N)
    mcol = mask.reshape(N, 1)
    ab = a.astype(bf16)
    tb = t.astype(bf16)
    vb = v.astype(bf16)

    eab, etb, evb, sqa, sqt, sqv = pl.pallas_call(
        _encode_kernel,
        out_shape=[jax.ShapeDtypeStruct((N, ED), bf16)] * 3
        + [jax.ShapeDtypeStruct((N, 1), f32)] * 3,
    )(ab, tb, vb, Wa_w, Wa_b.reshape(1, -1), Wt_w, Wt_b.reshape(1, -1),
      Wv_w, Wv_b.reshape(1, -1))

    aff = pl.pallas_call(
        _affinity_kernel,
        out_shape=jax.ShapeDtypeStruct((N, N), bf16),
    )
    pa = aff(eab, sqa, mrow, mcol)
    pt = aff(etb, sqt, mrow, mcol)
    pv = aff(evb, sqv, mrow, mcol)

    def fold(w, wb, d, db):
        dout = d.shape[0]
        return pl.pallas_call(
            _fold_kernel,
            out_shape=[jax.ShapeDtypeStruct((ED, dout), bf16),
                       jax.ShapeDtypeStruct((1, dout), f32)],
        )(w, d, wb.reshape(1, -1), db.reshape(1, -1))

    ma, b2a = fold(wa_w, wa_b, da_w, da_b)
    mt, b2t = fold(wt_w, wt_b, dt_w, dt_b)
    mv, b2v = fold(wv_w, wv_b, dv_w, dv_b)

    def headmix(eb, m, b2, p1, p2):
        return pl.pallas_call(
            _headmix_kernel,
            out_shape=jax.ShapeDtypeStruct((N, m.shape[1]), f32),
        )(eb, m, b2, p1, p2)

    ra = headmix(eab, ma, b2a, pt, pv)
    rt = headmix(etb, mt, b2t, pv, pa)
    rv = headmix(evb, mv, b2v, pa, pt)
    return jnp.concatenate([ra, rt, rv], axis=1)


# speedup vs baseline: 1.0150x; 1.0001x over previous
import jax
import jax.numpy as jnp
from jax import lax
from jax.experimental import pallas as pl

N = 1024
ED = 2048
K = 4
BIG = 1e30


def _bdotT(x, w):
    return lax.dot_general(x, w, (((1,), (1,)), ((), ())),
                           preferred_element_type=jnp.float32)


def _encode_kernel(a_ref, t_ref, v_ref, wa_ref, ba_ref, wt_ref, bt_ref,
                   wv_ref, bv_ref, eab_ref, etb_ref, evb_ref,
                   sqa_ref, sqt_ref, sqv_ref):
    ea = _bdotT(a_ref[...], wa_ref[...].astype(jnp.bfloat16)) + ba_ref[...]
    et = _bdotT(t_ref[...], wt_ref[...].astype(jnp.bfloat16)) + bt_ref[...]
    ev = _bdotT(v_ref[...], wv_ref[...].astype(jnp.bfloat16)) + bv_ref[...]
    eab_ref[...] = ea.astype(jnp.bfloat16)
    etb_ref[...] = et.astype(jnp.bfloat16)
    evb_ref[...] = ev.astype(jnp.bfloat16)
    sqa_ref[...] = jnp.sum(ea * ea, axis=1, keepdims=True)
    sqt_ref[...] = jnp.sum(et * et, axis=1, keepdims=True)
    sqv_ref[...] = jnp.sum(ev * ev, axis=1, keepdims=True)


def _affinity_kernel(x_ref, sq_ref, mrow_ref, mcol_ref, p_ref):
    sq = sq_ref[...]
    g = _bdotT(x_ref[...], x_ref[...])
    d2 = sq + sq.T - 2.0 * g
    iota = lax.broadcasted_iota(jnp.int32, (N, N), 1)
    rowi = lax.broadcasted_iota(jnp.int32, (N, N), 0)
    eye = iota == rowi
    jstars = []
    sims = []
    dwork = d2
    for _ in range(K):
        m = jnp.min(dwork, axis=1, keepdims=True)
        jstar = jnp.min(jnp.where(dwork == m, iota, N), axis=1, keepdims=True)
        jstars.append(jstar)
        sims.append(1.0 / (1.0 + jnp.sqrt(jnp.maximum(m, 0.0) + 1e-12)))
        dwork = jnp.where(iota == jstar, BIG, dwork)
    a_mat = jnp.zeros((N, N), jnp.float32)
    for jstar, sim in zip(jstars, sims):
        a_mat = a_mat + jnp.where(iota == jstar, sim, 0.0)
    a_mat = a_mat * mrow_ref[...] * mcol_ref[...]
    a_mat = jnp.maximum(a_mat, a_mat.T)
    s_mat = jnp.where(eye, 2.0, a_mat)
    dc = lax.rsqrt(jnp.sum(s_mat, axis=1, keepdims=True) + 1e-12)
    p_ref[...] = (dc * s_mat * dc.T).astype(jnp.bfloat16)


def _fold_kernel(w_ref, d_ref, wb_ref, db_ref, m_ref, b2_ref):
    dbf = d_ref[...].astype(jnp.bfloat16)
    m_ref[...] = lax.dot_general(
        w_ref[...].astype(jnp.bfloat16), dbf, (((0,), (1,)), ((), ())),
        preferred_element_type=jnp.float32).astype(jnp.bfloat16)
    b2_ref[...] = lax.dot_general(
        wb_ref[...].astype(jnp.bfloat16), dbf, (((1,), (1,)), ((), ())),
        preferred_element_type=jnp.float32) + db_ref[...]


def _headmix_kernel(e_ref, m_ref, b2_ref, p1_ref, p2_ref, o_ref):
    y = lax.dot_general(e_ref[...], m_ref[...], (((1,), (0,)), ((), ())),
                        preferred_element_type=jnp.float32)
    mixed = lax.dot_general(p1_ref[...] + p2_ref[...], y.astype(jnp.bfloat16),
                            (((1,), (0,)), ((), ())),
                            preferred_element_type=jnp.float32)
    o_ref[...] = 0.5 * y + 0.25 * mixed + b2_ref[...]


def kernel(a, t, v, mask, Wa_w, Wa_b, Wt_w, Wt_b, Wv_w, Wv_b,
           wa_w, wa_b, wt_w, wt_b, wv_w, wv_b,
           da_w, da_b, dt_w, dt_b, dv_w, dv_b):
    f32 = jnp.float32
    bf16 = jnp.bfloat16
    mrow = mask.reshape(1, N)
    mcol = mask.reshape(N, 1)
    ab = a.astype(bf16)
    tb = t.astype(bf16)
    vb = v.astype(bf16)

    eab, etb, evb, sqa, sqt, sqv = pl.pallas_call(
        _encode_kernel,
        out_shape=[jax.ShapeDtypeStruct((N, ED), bf16)] * 3
        + [jax.ShapeDtypeStruct((N, 1), f32)] * 3,
    )(ab, tb, vb, Wa_w, Wa_b.reshape(1, -1), Wt_w, Wt_b.reshape(1, -1),
      Wv_w, Wv_b.reshape(1, -1))

    aff = pl.pallas_call(
        _affinity_kernel,
        out_shape=jax.ShapeDtypeStruct((N, N), bf16),
    )
    pa = aff(eab, sqa, mrow, mcol)
    pt = aff(etb, sqt, mrow, mcol)
    pv = aff(evb, sqv, mrow, mcol)

    def fold(w, wb, d, db):
        dout = d.shape[0]
        return pl.pallas_call(
            _fold_kernel,
            out_shape=[jax.ShapeDtypeStruct((ED, dout), bf16),
                       jax.ShapeDtypeStruct((1, dout), f32)],
        )(w, d, wb.reshape(1, -1), db.reshape(1, -1))

    ma, b2a = fold(wa_w, wa_b, da_w, da_b)
    mt, b2t = fold(wt_w, wt_b, dt_w, dt_b)
    mv, b2v = fold(wv_w, wv_b, dv_w, dv_b)

    def headmix(eb, m, b2, p1, p2):
        return pl.pallas_call(
            _headmix_kernel,
            out_shape=jax.ShapeDtypeStruct((N, m.shape[1]), f32),
        )(eb, m, b2, p1, p2)

    ra = headmix(eab, ma, b2a, pt, pv)
    rt = headmix(etb, mt, b2t, pv, pa)
    rv = headmix(evb, mv, b2v, pa, pt)
    return jnp.concatenate([ra, rt, rv], axis=1)
